# baseline (speedup 1.0000x reference)
import functools
import numpy as np
import jax
import jax.numpy as jnp
from jax import lax
from jax.experimental import pallas as pl
from jax.experimental.pallas import tpu as pltpu
from jax.experimental.pallas import tpu_sc as plsc

_T = 64
_NS = 5
_SLOTS = _NS * _NS
_NH = 64
_SEL_PAD = 32

_PERM = np.array([7, 4, 16, 19, 2, 5, 3, 22, 6, 18, 10, 11, 15, 20, 8, 24,
                  9, 13, 14, 17, 23, 0, 21, 1, 12], dtype=np.int32)
_INV_PERM = np.argsort(_PERM).astype(np.int32)

_OUT_ROWS = 3 * _NS * _T * _NS
_ACTIVE_W = 30
_RPW = _OUT_ROWS // _ACTIVE_W


_NSPLIT = 4
_SW = 4096 // _NSPLIT
_RB = 8
_STEPS = _NH // _RB


def _scores_topk_body(*refs):
    (img_refs, (inv_perm_ref, img_any, sel_ref, cand_any, acc_ref,
                sem)) = refs[:_NSPLIT], refs[_NSPLIT:]
    i = pl.program_id(0)
    for j, img_ref in enumerate(img_refs):
        x = img_ref[...]
        for rb in range(_RB):
            x2 = x[:, rb * _T:(rb + 1) * _T, :].reshape(3 * _T, _SW)
            acc_ref[pl.ds(i * _RB + rb, 1), pl.ds(j * _SW, _SW)] = jnp.sum(
                x2, axis=0, keepdims=True)

    @pl.when(i == _STEPS - 1)
    def _():
        lane = lax.broadcasted_iota(jnp.int32, (4096, _NH), 0)
        col = lax.broadcasted_iota(jnp.int32, (4096, _NH), 1)
        S = (lane // _T == col).astype(jnp.float32)
        scores = jnp.dot(acc_ref[...], S,
                         precision=jax.lax.Precision.HIGHEST,
                         preferred_element_type=jnp.float32)
        ig = (lax.broadcasted_iota(jnp.int32, (_NH, _NH), 0) * _NH
              + lax.broadcasted_iota(jnp.int32, (_NH, _NH), 1))

        def body(k, s):
            m = jnp.max(s)
            idx = jnp.min(jnp.where(s == m, ig, jnp.int32(1 << 30)))
            sel_ref[inv_perm_ref[k]] = idx
            return jnp.where(ig == idx, jnp.float32(-jnp.inf), s)

        lax.fori_loop(0, _SLOTS, body, scores)
        for z in range(_SLOTS, _SEL_PAD):
            sel_ref[z] = 0

        copies = []
        for k in range(_SLOTS):
            s = sel_ref[k]
            ti = s // _NH
            tjc = lax.rem(s, _NH) // 2
            copies.append(pltpu.make_async_copy(
                img_any.at[:, pl.ds(ti * _T, _T), pl.ds(tjc * 2 * _T, 2 * _T)],
                cand_any.at[k],
                sem,
            ))
        for c in copies:
            c.start()
        for c in copies:
            c.wait()


def _select_tiles(img):
    inv_perm = jnp.asarray(_INV_PERM)
    return pl.pallas_call(
        _scores_topk_body,
        grid=(_STEPS,),
        in_specs=[
            pl.BlockSpec((3, _RB * _T, _SW), functools.partial(
                lambda j, i: (0, i, j), j))
            for j in range(_NSPLIT)
        ] + [
            pl.BlockSpec(memory_space=pltpu.SMEM),
            pl.BlockSpec(memory_space=pl.ANY),
        ],
        out_specs=[
            pl.BlockSpec(memory_space=pltpu.SMEM),
            pl.BlockSpec(memory_space=pl.ANY),
        ],
        out_shape=[
            jax.ShapeDtypeStruct((_SEL_PAD,), jnp.int32),
            jax.ShapeDtypeStruct((_SLOTS, 3, _T, 2 * _T), jnp.float32),
        ],
        scratch_shapes=[
            pltpu.VMEM((_NH, 4096), jnp.float32),
            pltpu.SemaphoreType.DMA,
        ],
    )(*([img] * _NSPLIT), inv_perm, img)


def _sc_gather_body(cand_hbm, sel_hbm, out_hbm, sel_v, idx0_v, idx1_v,
                    rows0_v, rows1_v, sem):
    info = plsc.get_sparse_core_info()
    wid = lax.axis_index("s") * info.num_cores + lax.axis_index("c")

    @pl.when(wid < _ACTIVE_W)
    def _():
        pltpu.sync_copy(sel_hbm, sel_v)
        base = wid * _RPW
        half = _RPW // 2
        for g in range(_RPW // 16):
            r = base + g * 16 + lax.iota(jnp.int32, 16)
            c = lax.div(r, 1600)
            rem = r - c * 1600
            irow = lax.div(rem, _NS)
            jcol = rem - irow * _NS
            islot = lax.div(irow, _T)
            kslot = islot * _NS + jcol
            a = irow - islot * _T
            selk = plsc.load_gather(sel_v, [kslot])
            tj = selk - lax.div(selk, _NH) * _NH
            h = tj - lax.div(tj, 2) * 2
            src = 2 * (kslot * (3 * _T) + c * _T + a) + h
            if g < half // 16:
                idx0_v[pl.ds(g * 16, 16)] = src
            else:
                idx1_v[pl.ds(g * 16 - half, 16)] = src
        cp0 = pltpu.async_copy(cand_hbm.at[idx0_v], rows0_v, sem)
        cp1 = pltpu.async_copy(cand_hbm.at[idx1_v], rows1_v, sem)
        cp0.wait()
        cp1.wait()
        pltpu.sync_copy(rows0_v, out_hbm.at[pl.ds(base, half)])
        pltpu.sync_copy(rows1_v, out_hbm.at[pl.ds(base + half, half)])


def _place_tiles(cand2, sel):
    mesh = plsc.VectorSubcoreMesh(core_axis_name="c", subcore_axis_name="s")
    f = functools.partial(
        pl.kernel,
        mesh=mesh,
        out_type=jax.ShapeDtypeStruct((_OUT_ROWS, _T), jnp.float32),
        compiler_params=pltpu.CompilerParams(
            use_tc_tiling_on_sc=False, needs_layout_passes=False),
        scratch_types=[
            pltpu.VMEM((_SEL_PAD,), jnp.int32),
            pltpu.VMEM((_RPW // 2,), jnp.int32),
            pltpu.VMEM((_RPW // 2,), jnp.int32),
            pltpu.VMEM((_RPW // 2, _T), jnp.float32),
            pltpu.VMEM((_RPW // 2, _T), jnp.float32),
            pltpu.SemaphoreType.DMA,
        ],
    )(_sc_gather_body)
    return f(cand2, sel)


def kernel(img):
    sel, cand = _select_tiles(img)
    cand2 = cand.reshape(_SLOTS * 3 * _T * 2, _T)
    out2 = _place_tiles(cand2, sel)
    return out2.reshape(3, _NS * _T, _NS * _T)


# speedup vs baseline: 1.0096x; 1.0096x over previous
import functools
import numpy as np
import jax
import jax.numpy as jnp
from jax import lax
from jax.experimental import pallas as pl
from jax.experimental.pallas import tpu as pltpu
from jax.experimental.pallas import tpu_sc as plsc

_T = 64
_NS = 5
_SLOTS = _NS * _NS
_NH = 64
_SEL_PAD = 32

_PERM = np.array([7, 4, 16, 19, 2, 5, 3, 22, 6, 18, 10, 11, 15, 20, 8, 24,
                  9, 13, 14, 17, 23, 0, 21, 1, 12], dtype=np.int32)
_INV_PERM = np.argsort(_PERM).astype(np.int32)

_OUT_ROWS = 3 * _NS * _T * _NS
_ACTIVE_W = 30
_RPW = _OUT_ROWS // _ACTIVE_W


_NSPLIT = 4
_SW = 4096 // _NSPLIT
_RB = 4
_STEPS = _NH // _RB


def _scores_topk_body(*refs):
    (img_refs, (inv_perm_ref, img_any, sel_ref, cand_any, acc_ref,
                sem)) = refs[:_NSPLIT], refs[_NSPLIT:]
    i = pl.program_id(0)
    for j, img_ref in enumerate(img_refs):
        x = img_ref[...]
        for rb in range(_RB):
            x2 = x[:, rb * _T:(rb + 1) * _T, :].reshape(3 * _T, _SW)
            acc_ref[pl.ds(i * _RB + rb, 1), pl.ds(j * _SW, _SW)] = jnp.sum(
                x2, axis=0, keepdims=True)

    @pl.when(i == _STEPS - 1)
    def _():
        lane = lax.broadcasted_iota(jnp.int32, (4096, _NH), 0)
        col = lax.broadcasted_iota(jnp.int32, (4096, _NH), 1)
        S = (lane // _T == col).astype(jnp.float32)
        scores = jnp.dot(acc_ref[...], S,
                         precision=jax.lax.Precision.HIGHEST,
                         preferred_element_type=jnp.float32)
        ig = (lax.broadcasted_iota(jnp.int32, (_NH, _NH), 0) * _NH
              + lax.broadcasted_iota(jnp.int32, (_NH, _NH), 1))

        def body(k, s):
            m = jnp.max(s)
            idx = jnp.min(jnp.where(s == m, ig, jnp.int32(1 << 30)))
            sel_ref[inv_perm_ref[k]] = idx
            return jnp.where(ig == idx, jnp.float32(-jnp.inf), s)

        lax.fori_loop(0, _SLOTS, body, scores)
        for z in range(_SLOTS, _SEL_PAD):
            sel_ref[z] = 0

        copies = []
        for k in range(_SLOTS):
            s = sel_ref[k]
            ti = s // _NH
            tjc = lax.rem(s, _NH) // 2
            copies.append(pltpu.make_async_copy(
                img_any.at[:, pl.ds(ti * _T, _T), pl.ds(tjc * 2 * _T, 2 * _T)],
                cand_any.at[k],
                sem,
            ))
        for c in copies:
            c.start()
        for c in copies:
            c.wait()


def _select_tiles(img):
    inv_perm = jnp.asarray(_INV_PERM)
    return pl.pallas_call(
        _scores_topk_body,
        grid=(_STEPS,),
        in_specs=[
            pl.BlockSpec((3, _RB * _T, _SW), functools.partial(
                lambda j, i: (0, i, j), j))
            for j in range(_NSPLIT)
        ] + [
            pl.BlockSpec(memory_space=pltpu.SMEM),
            pl.BlockSpec(memory_space=pl.ANY),
        ],
        out_specs=[
            pl.BlockSpec(memory_space=pltpu.SMEM),
            pl.BlockSpec(memory_space=pl.ANY),
        ],
        out_shape=[
            jax.ShapeDtypeStruct((_SEL_PAD,), jnp.int32),
            jax.ShapeDtypeStruct((_SLOTS, 3, _T, 2 * _T), jnp.float32),
        ],
        scratch_shapes=[
            pltpu.VMEM((_NH, 4096), jnp.float32),
            pltpu.SemaphoreType.DMA,
        ],
    )(*([img] * _NSPLIT), inv_perm, img)


def _sc_gather_body(cand_hbm, sel_hbm, out_hbm, sel_v, idx0_v, idx1_v,
                    rows0_v, rows1_v, sem):
    info = plsc.get_sparse_core_info()
    wid = lax.axis_index("s") * info.num_cores + lax.axis_index("c")

    @pl.when(wid < _ACTIVE_W)
    def _():
        pltpu.sync_copy(sel_hbm, sel_v)
        base = wid * _RPW
        half = _RPW // 2
        for g in range(_RPW // 16):
            r = base + g * 16 + lax.iota(jnp.int32, 16)
            c = lax.div(r, 1600)
            rem = r - c * 1600
            irow = lax.div(rem, _NS)
            jcol = rem - irow * _NS
            islot = lax.div(irow, _T)
            kslot = islot * _NS + jcol
            a = irow - islot * _T
            selk = plsc.load_gather(sel_v, [kslot])
            tj = selk - lax.div(selk, _NH) * _NH
            h = tj - lax.div(tj, 2) * 2
            src = 2 * (kslot * (3 * _T) + c * _T + a) + h
            if g < half // 16:
                idx0_v[pl.ds(g * 16, 16)] = src
            else:
                idx1_v[pl.ds(g * 16 - half, 16)] = src
        cp0 = pltpu.async_copy(cand_hbm.at[idx0_v], rows0_v, sem)
        cp1 = pltpu.async_copy(cand_hbm.at[idx1_v], rows1_v, sem)
        cp0.wait()
        cp1.wait()
        pltpu.sync_copy(rows0_v, out_hbm.at[pl.ds(base, half)])
        pltpu.sync_copy(rows1_v, out_hbm.at[pl.ds(base + half, half)])


def _place_tiles(cand2, sel):
    mesh = plsc.VectorSubcoreMesh(core_axis_name="c", subcore_axis_name="s")
    f = functools.partial(
        pl.kernel,
        mesh=mesh,
        out_type=jax.ShapeDtypeStruct((_OUT_ROWS, _T), jnp.float32),
        compiler_params=pltpu.CompilerParams(
            use_tc_tiling_on_sc=False, needs_layout_passes=False),
        scratch_types=[
            pltpu.VMEM((_SEL_PAD,), jnp.int32),
            pltpu.VMEM((_RPW // 2,), jnp.int32),
            pltpu.VMEM((_RPW // 2,), jnp.int32),
            pltpu.VMEM((_RPW // 2, _T), jnp.float32),
            pltpu.VMEM((_RPW // 2, _T), jnp.float32),
            pltpu.SemaphoreType.DMA,
        ],
    )(_sc_gather_body)
    return f(cand2, sel)


def kernel(img):
    sel, cand = _select_tiles(img)
    cand2 = cand.reshape(_SLOTS * 3 * _T * 2, _T)
    out2 = _place_tiles(cand2, sel)
    return out2.reshape(3, _NS * _T, _NS * _T)
